# Initial kernel scaffold; baseline (speedup 1.0000x reference)
import jax
import jax.numpy as jnp
from jax.experimental import pallas as pl


def kernel(x, edge_index, W1, Wres1, gamma1, beta1, W2, Wres2, gamma2, beta2):
    raise NotImplementedError("write your pallas kernel here")


# speedup vs baseline: 4.6049x; 4.6049x over previous
import functools

import jax
import jax.numpy as jnp
from jax import lax
from jax.experimental import pallas as pl
from jax.experimental.pallas import tpu as pltpu
from jax.experimental.pallas import tpu_sc as plsc

N = 10000
D = 128
E = 320000
NC = 2
NS = 16
NW = NC * NS
EPW = E // NW
CHUNK = 80
NCHUNK = EPW // CHUNK
RPT = 624
ZR = 16

BLK = 1000
NB = N // BLK


def _seg_sum_body(x_hbm, src_hbm, dst_hbm, out_hbm,
                  src_v, dst_v, rows_v, zbuf, acc, sem):
    c = lax.axis_index("c")
    s = lax.axis_index("s")
    wid = s * NC + c

    def zb(i, _):
        r = i // 8
        col = (i % 8) * 16
        zbuf[r, pl.ds(col, 16)] = jnp.zeros((16,), jnp.float32)
        return 0
    lax.fori_loop(0, ZR * 8, zb, 0)

    row0 = s * RPT
    nz = jnp.where(s == NS - 1, (RPT + 16) // ZR, RPT // ZR)

    def zc(i, _):
        pltpu.sync_copy(zbuf, acc.at[pl.ds(pl.multiple_of(row0 + i * ZR, 8), ZR)])
        return 0
    lax.fori_loop(0, nz, zc, 0)
    plsc.subcore_barrier()

    base0 = wid * EPW

    def body(i, _):
        base = pl.multiple_of(base0 + i * CHUNK, 8)
        pltpu.sync_copy(src_hbm.at[pl.ds(base, CHUNK)], src_v)
        pltpu.sync_copy(dst_hbm.at[pl.ds(base, CHUNK)], dst_v)
        pltpu.async_copy(x_hbm.at[src_v], rows_v, sem).wait()
        pltpu.sync_copy(rows_v, acc.at[dst_v], add=True)
        return 0
    lax.fori_loop(0, NCHUNK, body, 0)
    plsc.subcore_barrier()

    pltpu.sync_copy(acc.at[pl.ds(row0, RPT)], out_hbm.at[c, pl.ds(row0, RPT)])

    @pl.when(s == NS - 1)
    def _():
        pltpu.sync_copy(acc.at[pl.ds(N - 16, 16)],
                        out_hbm.at[c, pl.ds(N - 16, 16)])


@functools.lru_cache(maxsize=None)
def _seg_sum_call():
    return pl.kernel(
        _seg_sum_body,
        out_type=jax.ShapeDtypeStruct((NC, N, D), jnp.float32),
        mesh=plsc.VectorSubcoreMesh(core_axis_name="c", subcore_axis_name="s"),
        scratch_types=[
            pltpu.VMEM((CHUNK,), jnp.int32),
            pltpu.VMEM((CHUNK,), jnp.int32),
            pltpu.VMEM((CHUNK, D), jnp.float32),
            pltpu.VMEM((ZR, D), jnp.float32),
            pltpu.VMEM_SHARED((N, D), jnp.float32),
            pltpu.SemaphoreType.DMA,
        ],
    )


def _dense_body(p_ref, x_ref, w_ref, wres_ref, h_ref, stats_ref):
    i = pl.program_id(0)
    agg = p_ref[0] + p_ref[1]
    h = jnp.maximum(jnp.dot(agg, w_ref[...],
                            preferred_element_type=jnp.float32), 0.0)
    res = jnp.maximum(jnp.dot(x_ref[...], wres_ref[...],
                              preferred_element_type=jnp.float32), 0.0)
    h = h + res
    h_ref[...] = h
    bsum = jnp.sum(h, axis=0, keepdims=True)
    bsq = jnp.sum(h * h, axis=0, keepdims=True)
    blk = jnp.concatenate([bsum, bsq], axis=0)

    @pl.when(i == 0)
    def _():
        stats_ref[...] = jnp.zeros_like(stats_ref)
    stats_ref[...] += blk


@functools.lru_cache(maxsize=None)
def _dense_call():
    return pl.pallas_call(
        _dense_body,
        grid=(NB,),
        in_specs=[
            pl.BlockSpec((NC, BLK, D), lambda i: (0, i, 0)),
            pl.BlockSpec((BLK, D), lambda i: (i, 0)),
            pl.BlockSpec((D, D), lambda i: (0, 0)),
            pl.BlockSpec((D, D), lambda i: (0, 0)),
        ],
        out_specs=[
            pl.BlockSpec((BLK, D), lambda i: (i, 0)),
            pl.BlockSpec((2, D), lambda i: (0, 0)),
        ],
        out_shape=[
            jax.ShapeDtypeStruct((N, D), jnp.float32),
            jax.ShapeDtypeStruct((2, D), jnp.float32),
        ],
    )


def _norm_body(h_ref, stats_ref, g_ref, b_ref, out_ref):
    mean = stats_ref[0:1] * (1.0 / N)
    var = stats_ref[1:2] * (1.0 / N) - mean * mean
    inv = lax.rsqrt(var + 1e-5)
    out_ref[...] = (h_ref[...] - mean) * (inv * g_ref[...]) + b_ref[...]


@functools.lru_cache(maxsize=None)
def _norm_call():
    return pl.pallas_call(
        _norm_body,
        grid=(NB,),
        in_specs=[
            pl.BlockSpec((BLK, D), lambda i: (i, 0)),
            pl.BlockSpec((2, D), lambda i: (0, 0)),
            pl.BlockSpec((1, D), lambda i: (0, 0)),
            pl.BlockSpec((1, D), lambda i: (0, 0)),
        ],
        out_specs=pl.BlockSpec((BLK, D), lambda i: (i, 0)),
        out_shape=jax.ShapeDtypeStruct((N, D), jnp.float32),
    )


def _layer(x, src, dst, w, wres, gamma, beta):
    p = _seg_sum_call()(x, src, dst)
    h, stats = _dense_call()(p, x, w, wres)
    return _norm_call()(h, stats, gamma.reshape(1, D), beta.reshape(1, D))


def kernel(x, edge_index, W1, Wres1, gamma1, beta1, W2, Wres2, gamma2, beta2):
    src = edge_index[0]
    dst = edge_index[1]
    h = _layer(x, src, dst, W1, Wres1, gamma1, beta1)
    h = _layer(h, src, dst, W2, Wres2, gamma2, beta2)
    return h


# speedup vs baseline: 8.2694x; 1.7958x over previous
import functools

import jax
import jax.numpy as jnp
from jax import lax
from jax.experimental import pallas as pl
from jax.experimental.pallas import tpu as pltpu
from jax.experimental.pallas import tpu_sc as plsc

N = 10000
D = 128
E = 320000
NC = 2
NS = 16
NW = NC * NS
EPW = E // NW
CHUNK = 80
NCHUNK = EPW // CHUNK
RPT = 624
ZR = 16

BLK = 1000
NB = N // BLK


def _seg_sum_body(x_hbm, src_hbm, dst_hbm, out_hbm,
                  idx_s, idx_d, rows, zbuf, acc,
                  sem_ip, sem_g0, sem_g1, sem_s0, sem_s1):
    c = lax.axis_index("c")
    s = lax.axis_index("s")
    wid = s * NC + c

    ip_s = pltpu.async_copy(
        src_hbm.at[pl.ds(pl.multiple_of(wid * EPW, 8), EPW)], idx_s, sem_ip)
    ip_d = pltpu.async_copy(dst_hbm.at[wid], idx_d, sem_ip)

    def zb(i, _):
        r = i // 8
        col = (i % 8) * 16
        zbuf[r, pl.ds(col, 16)] = jnp.zeros((16,), jnp.float32)
        return 0
    lax.fori_loop(0, ZR * 8, zb, 0)

    row0 = s * RPT
    nz = jnp.where(s == NS - 1, (RPT + 16) // ZR, RPT // ZR)

    def zc(i, _):
        pltpu.sync_copy(zbuf, acc.at[pl.ds(pl.multiple_of(row0 + i * ZR, 8), ZR)])
        return 0
    lax.fori_loop(0, nz, zc, 0)
    ip_s.wait()
    ip_d.wait()
    plsc.subcore_barrier()

    sem_g = (sem_g0, sem_g1)
    sem_s = (sem_s0, sem_s1)

    def gather_start(j, b):
        pltpu.async_copy(x_hbm.at[idx_s.at[pl.ds(j * CHUNK, CHUNK)]],
                         rows.at[b], sem_g[b])

    def gather_wait(j, b):
        pltpu.make_async_copy(x_hbm.at[idx_s.at[pl.ds(j * CHUNK, CHUNK)]],
                              rows.at[b], sem_g[b]).wait()

    def scatter_start(j, b):
        pltpu.async_copy(rows.at[b], acc.at[idx_d.at[j]], sem_s[b], add=True)

    def scatter_wait(j, b):
        pltpu.make_async_copy(rows.at[b], acc.at[idx_d.at[j]],
                              sem_s[b]).wait()

    gather_start(0, 0)

    def body(i, _):
        j0 = i * 2
        j1 = j0 + 1
        gather_wait(j0, 0)

        @pl.when(i > 0)
        def _():
            scatter_wait(j0 - 1, 1)
        gather_start(j1, 1)
        scatter_start(j0, 0)
        gather_wait(j1, 1)
        scatter_wait(j0, 0)
        gather_start(j0 + 2, 0)
        scatter_start(j1, 1)
        return 0
    lax.fori_loop(0, NCHUNK // 2, body, 0)
    scatter_wait(NCHUNK - 2, 1)
    gather_wait(NCHUNK - 1, 0)
    scatter_start(NCHUNK - 1, 0)
    scatter_wait(NCHUNK - 1, 0)
    plsc.subcore_barrier()

    pltpu.sync_copy(acc.at[pl.ds(row0, RPT)], out_hbm.at[c, pl.ds(row0, RPT)])

    @pl.when(s == NS - 1)
    def _():
        pltpu.sync_copy(acc.at[pl.ds(N - 16, 16)],
                        out_hbm.at[c, pl.ds(N - 16, 16)])


@functools.lru_cache(maxsize=None)
def _seg_sum_call():
    return pl.kernel(
        _seg_sum_body,
        out_type=jax.ShapeDtypeStruct((NC, N, D), jnp.float32),
        mesh=plsc.VectorSubcoreMesh(core_axis_name="c", subcore_axis_name="s"),
        scratch_types=[
            pltpu.VMEM((EPW,), jnp.int32),
            pltpu.VMEM((NCHUNK, CHUNK), jnp.int32),
            pltpu.VMEM((2, CHUNK, D), jnp.float32),
            pltpu.VMEM((ZR, D), jnp.float32),
            pltpu.VMEM_SHARED((N, D), jnp.float32),
            pltpu.SemaphoreType.DMA,
            pltpu.SemaphoreType.DMA,
            pltpu.SemaphoreType.DMA,
            pltpu.SemaphoreType.DMA,
            pltpu.SemaphoreType.DMA,
        ],
    )


def _dense_body(p_ref, x_ref, w_ref, wres_ref, h_ref, stats_ref):
    i = pl.program_id(0)
    agg = p_ref[0] + p_ref[1]
    h = jnp.maximum(jnp.dot(agg, w_ref[...],
                            preferred_element_type=jnp.float32), 0.0)
    res = jnp.maximum(jnp.dot(x_ref[...], wres_ref[...],
                              preferred_element_type=jnp.float32), 0.0)
    h = h + res
    h_ref[...] = h
    bsum = jnp.sum(h, axis=0, keepdims=True)
    bsq = jnp.sum(h * h, axis=0, keepdims=True)
    blk = jnp.concatenate([bsum, bsq], axis=0)

    @pl.when(i == 0)
    def _():
        stats_ref[...] = jnp.zeros_like(stats_ref)
    stats_ref[...] += blk


@functools.lru_cache(maxsize=None)
def _dense_call():
    return pl.pallas_call(
        _dense_body,
        grid=(NB,),
        in_specs=[
            pl.BlockSpec((NC, BLK, D), lambda i: (0, i, 0)),
            pl.BlockSpec((BLK, D), lambda i: (i, 0)),
            pl.BlockSpec((D, D), lambda i: (0, 0)),
            pl.BlockSpec((D, D), lambda i: (0, 0)),
        ],
        out_specs=[
            pl.BlockSpec((BLK, D), lambda i: (i, 0)),
            pl.BlockSpec((2, D), lambda i: (0, 0)),
        ],
        out_shape=[
            jax.ShapeDtypeStruct((N, D), jnp.float32),
            jax.ShapeDtypeStruct((2, D), jnp.float32),
        ],
    )


def _norm_body(h_ref, stats_ref, g_ref, b_ref, out_ref):
    mean = stats_ref[0:1] * (1.0 / N)
    var = stats_ref[1:2] * (1.0 / N) - mean * mean
    inv = lax.rsqrt(var + 1e-5)
    out_ref[...] = (h_ref[...] - mean) * (inv * g_ref[...]) + b_ref[...]


@functools.lru_cache(maxsize=None)
def _norm_call():
    return pl.pallas_call(
        _norm_body,
        grid=(NB,),
        in_specs=[
            pl.BlockSpec((BLK, D), lambda i: (i, 0)),
            pl.BlockSpec((2, D), lambda i: (0, 0)),
            pl.BlockSpec((1, D), lambda i: (0, 0)),
            pl.BlockSpec((1, D), lambda i: (0, 0)),
        ],
        out_specs=pl.BlockSpec((BLK, D), lambda i: (i, 0)),
        out_shape=jax.ShapeDtypeStruct((N, D), jnp.float32),
    )


def _layer(x, src, dst, w, wres, gamma, beta):
    p = _seg_sum_call()(x, src, dst)
    h, stats = _dense_call()(p, x, w, wres)
    return _norm_call()(h, stats, gamma.reshape(1, D), beta.reshape(1, D))


def kernel(x, edge_index, W1, Wres1, gamma1, beta1, W2, Wres2, gamma2, beta2):
    src = edge_index[0]
    dst = edge_index[1].reshape(NW, NCHUNK, CHUNK)
    h = _layer(x, src, dst, W1, Wres1, gamma1, beta1)
    h = _layer(h, src, dst, W2, Wres2, gamma2, beta2)
    return h


# speedup vs baseline: 9.0242x; 1.0913x over previous
import functools

import jax
import jax.numpy as jnp
from jax import lax
from jax.experimental import pallas as pl
from jax.experimental.pallas import tpu as pltpu
from jax.experimental.pallas import tpu_sc as plsc

N = 10000
D = 128
E = 320000
NC = 2
NS = 16
NW = NC * NS
EPW = E // NW
CHUNK = 104
NCHUNK = EPW // CHUNK
TAIL = EPW - NCHUNK * CHUNK
RPT = 624
ZR = 16

BLK = 1000
NB = N // BLK


def _seg_sum_body(x_hbm, src_hbm, dstm_hbm, dstt_hbm, out_hbm,
                  idx_s, idx_d, idx_dt, rows, acc,
                  sem_ip, sem_g0, sem_g1, sem_s0, sem_s1):
    c = lax.axis_index("c")
    s = lax.axis_index("s")
    wid = s * NC + c

    ip_s = pltpu.async_copy(
        src_hbm.at[pl.ds(pl.multiple_of(wid * EPW, 8), EPW)], idx_s, sem_ip)
    ip_d = pltpu.async_copy(dstm_hbm.at[wid], idx_d, sem_ip)
    ip_t = pltpu.async_copy(dstt_hbm.at[wid], idx_dt, sem_ip)

    def zb(i, _):
        r = i // 8
        col = (i % 8) * 16
        rows[0, r, pl.ds(col, 16)] = jnp.zeros((16,), jnp.float32)
        return 0
    lax.fori_loop(0, ZR * 8, zb, 0)

    row0 = s * RPT
    nz = jnp.where(s == NS - 1, (RPT + 16) // ZR, RPT // ZR)

    def zc(i, _):
        pltpu.sync_copy(rows.at[0, pl.ds(0, ZR)],
                        acc.at[pl.ds(pl.multiple_of(row0 + i * ZR, 8), ZR)])
        return 0
    lax.fori_loop(0, nz, zc, 0)
    ip_s.wait()
    ip_d.wait()
    ip_t.wait()
    plsc.subcore_barrier()

    sem_g = (sem_g0, sem_g1)
    sem_s = (sem_s0, sem_s1)

    def gather_start(j, b):
        pltpu.async_copy(x_hbm.at[idx_s.at[pl.ds(j * CHUNK, CHUNK)]],
                         rows.at[b], sem_g[b])

    def gather_wait(j, b):
        pltpu.make_async_copy(x_hbm.at[idx_s.at[pl.ds(j * CHUNK, CHUNK)]],
                              rows.at[b], sem_g[b]).wait()

    def scatter_start(j, b):
        pltpu.async_copy(rows.at[b], acc.at[idx_d.at[j]], sem_s[b], add=True)

    def scatter_wait(j, b):
        pltpu.make_async_copy(rows.at[b], acc.at[idx_d.at[j]],
                              sem_s[b]).wait()

    gather_start(0, 0)

    def body(i, _):
        j0 = i * 2
        j1 = j0 + 1
        gather_wait(j0, 0)

        @pl.when(i > 0)
        def _():
            scatter_wait(j0 - 1, 1)
        gather_start(j1, 1)
        scatter_start(j0, 0)
        gather_wait(j1, 1)
        scatter_wait(j0, 0)

        @pl.when(j0 + 2 < NCHUNK)
        def _():
            gather_start(j0 + 2, 0)
        scatter_start(j1, 1)
        return 0
    lax.fori_loop(0, NCHUNK // 2, body, 0)
    t0 = pl.multiple_of(NCHUNK * CHUNK, 8)
    pltpu.async_copy(x_hbm.at[idx_s.at[pl.ds(t0, TAIL)]],
                     rows.at[0, pl.ds(0, TAIL)], sem_g0)
    pltpu.make_async_copy(x_hbm.at[idx_s.at[pl.ds(t0, TAIL)]],
                          rows.at[0, pl.ds(0, TAIL)], sem_g0).wait()
    pltpu.async_copy(rows.at[0, pl.ds(0, TAIL)], acc.at[idx_dt.at[0]],
                     sem_s0, add=True)
    pltpu.make_async_copy(rows.at[0, pl.ds(0, TAIL)], acc.at[idx_dt.at[0]],
                          sem_s0).wait()
    scatter_wait(NCHUNK - 1, 1)
    plsc.subcore_barrier()

    pltpu.sync_copy(acc.at[pl.ds(row0, RPT)], out_hbm.at[c, pl.ds(row0, RPT)])

    @pl.when(s == NS - 1)
    def _():
        pltpu.sync_copy(acc.at[pl.ds(N - 16, 16)],
                        out_hbm.at[c, pl.ds(N - 16, 16)])


@functools.lru_cache(maxsize=None)
def _seg_sum_call():
    return pl.kernel(
        _seg_sum_body,
        out_type=jax.ShapeDtypeStruct((NC, N, D), jnp.float32),
        mesh=plsc.VectorSubcoreMesh(core_axis_name="c", subcore_axis_name="s"),
        scratch_types=[
            pltpu.VMEM((EPW,), jnp.int32),
            pltpu.VMEM((NCHUNK, CHUNK), jnp.int32),
            pltpu.VMEM((1, TAIL), jnp.int32),
            pltpu.VMEM((2, CHUNK, D), jnp.float32),
            pltpu.VMEM_SHARED((N, D), jnp.float32),
            pltpu.SemaphoreType.DMA,
            pltpu.SemaphoreType.DMA,
            pltpu.SemaphoreType.DMA,
            pltpu.SemaphoreType.DMA,
            pltpu.SemaphoreType.DMA,
        ],
    )


def _dense_body(p_ref, x_ref, w_ref, wres_ref, h_ref, stats_ref):
    i = pl.program_id(0)
    agg = p_ref[0] + p_ref[1]
    h = jnp.maximum(jnp.dot(agg, w_ref[...],
                            preferred_element_type=jnp.float32), 0.0)
    res = jnp.maximum(jnp.dot(x_ref[...], wres_ref[...],
                              preferred_element_type=jnp.float32), 0.0)
    h = h + res
    h_ref[...] = h
    bsum = jnp.sum(h, axis=0, keepdims=True)
    bsq = jnp.sum(h * h, axis=0, keepdims=True)
    blk = jnp.concatenate([bsum, bsq], axis=0)

    @pl.when(i == 0)
    def _():
        stats_ref[...] = jnp.zeros_like(stats_ref)
    stats_ref[...] += blk


@functools.lru_cache(maxsize=None)
def _dense_call():
    return pl.pallas_call(
        _dense_body,
        grid=(NB,),
        in_specs=[
            pl.BlockSpec((NC, BLK, D), lambda i: (0, i, 0)),
            pl.BlockSpec((BLK, D), lambda i: (i, 0)),
            pl.BlockSpec((D, D), lambda i: (0, 0)),
            pl.BlockSpec((D, D), lambda i: (0, 0)),
        ],
        out_specs=[
            pl.BlockSpec((BLK, D), lambda i: (i, 0)),
            pl.BlockSpec((2, D), lambda i: (0, 0)),
        ],
        out_shape=[
            jax.ShapeDtypeStruct((N, D), jnp.float32),
            jax.ShapeDtypeStruct((2, D), jnp.float32),
        ],
    )


def _norm_body(h_ref, stats_ref, g_ref, b_ref, out_ref):
    mean = stats_ref[0:1] * (1.0 / N)
    var = stats_ref[1:2] * (1.0 / N) - mean * mean
    inv = lax.rsqrt(var + 1e-5)
    out_ref[...] = (h_ref[...] - mean) * (inv * g_ref[...]) + b_ref[...]


@functools.lru_cache(maxsize=None)
def _norm_call():
    return pl.pallas_call(
        _norm_body,
        grid=(NB,),
        in_specs=[
            pl.BlockSpec((BLK, D), lambda i: (i, 0)),
            pl.BlockSpec((2, D), lambda i: (0, 0)),
            pl.BlockSpec((1, D), lambda i: (0, 0)),
            pl.BlockSpec((1, D), lambda i: (0, 0)),
        ],
        out_specs=pl.BlockSpec((BLK, D), lambda i: (i, 0)),
        out_shape=jax.ShapeDtypeStruct((N, D), jnp.float32),
    )


def _layer(x, src, dst_m, dst_t, w, wres, gamma, beta):
    p = _seg_sum_call()(x, src, dst_m, dst_t)
    h, stats = _dense_call()(p, x, w, wres)
    return _norm_call()(h, stats, gamma.reshape(1, D), beta.reshape(1, D))


def kernel(x, edge_index, W1, Wres1, gamma1, beta1, W2, Wres2, gamma2, beta2):
    src = edge_index[0]
    dst2 = edge_index[1].reshape(NW, EPW)
    dst_m = dst2[:, :NCHUNK * CHUNK].reshape(NW, NCHUNK, CHUNK)
    dst_t = dst2[:, NCHUNK * CHUNK:].reshape(NW, 1, TAIL)
    h = _layer(x, src, dst_m, dst_t, W1, Wres1, gamma1, beta1)
    h = _layer(h, src, dst_m, dst_t, W2, Wres2, gamma2, beta2)
    return h
